# baseline (speedup 1.0000x reference)
import functools

import jax
import jax.numpy as jnp
from jax import lax
from jax.experimental import pallas as pl
from jax.experimental.pallas import tpu as pltpu
from jax.experimental.pallas import tpu_sc as plsc

N_NODE = 10000
E = 320000
NC = 2
NS = 16
ICH = 4000
NICH = E // ICH
GU = 2
ZR = 2000
BM = 2000


def _tc_mm4(xp, xa, wsp, wrap, wsa, wrpa):
    Mrows, K = xp.shape
    N = wsp.shape[1]

    def body(xp_ref, xa_ref, wsp_ref, wrap_ref, wsa_ref, wrpa_ref,
             osp, oxwap, osa, oxwpa):
        xp_b = xp_ref[...]
        xa_b = xa_ref[...]
        osp[...] = jnp.dot(xp_b, wsp_ref[...], preferred_element_type=jnp.float32)
        oxwap[...] = jnp.dot(xp_b, wrap_ref[...], preferred_element_type=jnp.float32)
        osa[...] = jnp.dot(xa_b, wsa_ref[...], preferred_element_type=jnp.float32)
        oxwpa[...] = jnp.dot(xa_b, wrpa_ref[...], preferred_element_type=jnp.float32)

    bs_x = pl.BlockSpec((BM, K), lambda i: (i, 0))
    bs_w = pl.BlockSpec((K, N), lambda i: (0, 0))
    bs_o = pl.BlockSpec((BM, N), lambda i: (i, 0))
    return pl.pallas_call(
        body, grid=(Mrows // BM,),
        in_specs=[bs_x, bs_x, bs_w, bs_w, bs_w, bs_w],
        out_specs=[bs_o, bs_o, bs_o, bs_o],
        out_shape=[jax.ShapeDtypeStruct((Mrows, N), jnp.float32)] * 4,
    )(xp, xa, wsp, wrap, wsa, wrpa)


def _tc_comb_mm4(sp, nbp, bp, sa, nba, ba, wsp, wrap, wsa, wrpa):
    Mrows, K = sp.shape
    N = wsp.shape[1]

    def body(sp_ref, nbp_ref, bp_ref, sa_ref, nba_ref, ba_ref,
             wsp_ref, wrap_ref, wsa_ref, wrpa_ref,
             osp, oxwap, osa, oxwpa):
        xp = (sp_ref[...] + nbp_ref[...]) * 0.5 + bp_ref[...]
        xp = jnp.where(xp > 0, xp, jnp.exp(jnp.minimum(xp, 0.0)) - 1.0)
        xa = (sa_ref[...] + nba_ref[...]) * 0.5 + ba_ref[...]
        xa = jnp.where(xa > 0, xa, jnp.exp(jnp.minimum(xa, 0.0)) - 1.0)
        osp[...] = jnp.dot(xp, wsp_ref[...], preferred_element_type=jnp.float32)
        oxwap[...] = jnp.dot(xp, wrap_ref[...], preferred_element_type=jnp.float32)
        osa[...] = jnp.dot(xa, wsa_ref[...], preferred_element_type=jnp.float32)
        oxwpa[...] = jnp.dot(xa, wrpa_ref[...], preferred_element_type=jnp.float32)

    bs_x = pl.BlockSpec((BM, K), lambda i: (i, 0))
    bs_b = pl.BlockSpec((1, K), lambda i: (0, 0))
    bs_w = pl.BlockSpec((K, N), lambda i: (0, 0))
    bs_o = pl.BlockSpec((BM, N), lambda i: (i, 0))
    return pl.pallas_call(
        body, grid=(Mrows // BM,),
        in_specs=[bs_x, bs_x, bs_b, bs_x, bs_x, bs_b,
                  bs_w, bs_w, bs_w, bs_w],
        out_specs=[bs_o, bs_o, bs_o, bs_o],
        out_shape=[jax.ShapeDtypeStruct((Mrows, N), jnp.float32)] * 4,
    )(sp, nbp, bp, sa, nba, ba, wsp, wrap, wsa, wrpa)


def _tc_final(sp, nbp, bp, sa, nba, ba):
    Mrows, K = sp.shape

    def body(sp_ref, nbp_ref, bp_ref, sa_ref, nba_ref, ba_ref, op, oa):
        op[...] = (sp_ref[...] + nbp_ref[...]) * 0.5 + bp_ref[...]
        oa[...] = (sa_ref[...] + nba_ref[...]) * 0.5 + ba_ref[...]

    bs_x = pl.BlockSpec((BM, K), lambda i: (i, 0))
    bs_b = pl.BlockSpec((1, K), lambda i: (0, 0))
    return pl.pallas_call(
        body, grid=(Mrows // BM,),
        in_specs=[bs_x, bs_x, bs_b, bs_x, bs_x, bs_b],
        out_specs=[bs_x, bs_x],
        out_shape=[jax.ShapeDtypeStruct((Mrows, K), jnp.float32)] * 2,
    )(sp, nbp, bp, sa, nba, ba)


def _sc_spmm(tbl_p, tbl_a, src_pa, dst_pa, src_ap, dst_ap, zeros):
    d = tbl_p.shape[1]
    CG = d // NS
    tbl_p = jnp.transpose(tbl_p.reshape(N_NODE, NS, CG), (1, 0, 2))
    tbl_p = tbl_p.reshape(NS, N_NODE * CG)
    tbl_a = jnp.transpose(tbl_a.reshape(N_NODE, NS, CG), (1, 0, 2))
    tbl_a = tbl_a.reshape(NS, N_NODE * CG)
    mesh = plsc.VectorSubcoreMesh(core_axis_name="c", subcore_axis_name="s")
    out_t = (jax.ShapeDtypeStruct((NS, N_NODE * CG), jnp.float32),
             jax.ShapeDtypeStruct((NS, N_NODE * CG), jnp.float32))

    @functools.partial(
        pl.kernel, mesh=mesh, out_type=out_t,
        compiler_params=pltpu.CompilerParams(
            use_tc_tiling_on_sc=False, needs_layout_passes=False),
        scratch_types=[
            pltpu.VMEM((N_NODE * CG,), jnp.float32),
            pltpu.VMEM((N_NODE * CG,), jnp.float32),
            pltpu.VMEM((2, ICH), jnp.int32),
            pltpu.VMEM((2, ICH), jnp.int32),
            pltpu.SemaphoreType.DMA((2,)),
            pltpu.SemaphoreType.DMA((2,)),
        ],
    )
    def k(tblp_h, tbla_h, spa_h, dpa_h, sap_h, dap_h, zeros_h,
          outp_h, outa_h, tblbuf, accbuf, ibs, ibd, ssem, dsem):
        c = lax.axis_index("c")
        s = lax.axis_index("s")
        zch = ZR * CG
        for kz in range(N_NODE // ZR):
            pltpu.sync_copy(zeros_h, accbuf.at[pl.ds(kz * zch, zch)])

        def run(src_h, dst_h, tbl_h, out_h):
            pltpu.sync_copy(tbl_h.at[s], tblbuf)
            for b in range(2):
                pltpu.async_copy(src_h.at[b], ibs.at[b], ssem.at[b])
                pltpu.async_copy(dst_h.at[b], ibd.at[b], dsem.at[b])
            def outer(t2, _):
                for b in range(2):
                    t = t2 * 2 + b
                    pltpu.make_async_copy(
                        src_h.at[t], ibs.at[b], ssem.at[b]).wait()
                    pltpu.make_async_copy(
                        dst_h.at[t], ibd.at[b], dsem.at[b]).wait()

                    def grp(g, _, b=b):
                        for u in range(GU):
                            base = (g * GU + u) * 16
                            srcf = ibs[b, pl.ds(base, 16)] * CG
                            dstf = ibd[b, pl.ds(base, 16)] * CG
                            for j in range(CG):
                                v = plsc.load_gather(tblbuf, [srcf + j])
                                plsc.addupdate_scatter(accbuf, [dstf + j], v)
                        return 0

                    lax.fori_loop(0, ICH // 16 // GU, grp, 0)

                    @pl.when(t + 2 < NICH)
                    def _(b=b, t=t):
                        pltpu.async_copy(src_h.at[t + 2], ibs.at[b], ssem.at[b])
                        pltpu.async_copy(dst_h.at[t + 2], ibd.at[b], dsem.at[b])
                return 0

            lax.fori_loop(0, NICH // 2, outer, 0)
            pltpu.sync_copy(accbuf, out_h.at[s])

        @pl.when(c == 0)
        def _():
            run(spa_h, dpa_h, tblp_h, outp_h)

        @pl.when(c == 1)
        def _():
            run(sap_h, dap_h, tbla_h, outa_h)

    nbp, nba = k(tbl_p, tbl_a, src_pa, dst_pa, src_ap, dst_ap, zeros)
    nbp = jnp.transpose(nbp.reshape(NS, N_NODE, CG), (1, 0, 2)).reshape(N_NODE, d)
    nba = jnp.transpose(nba.reshape(NS, N_NODE, CG), (1, 0, 2)).reshape(N_NODE, d)
    return nbp, nba


def kernel(ft_p, ft_a, adj_p_a, adj_a_p,
           w_self_p_0, w_rel_p_a_0, bias_p_0, w_self_a_0, w_rel_a_p_0, bias_a_0,
           w_self_p_1, w_rel_p_a_1, bias_p_1, w_self_a_1, w_rel_a_p_1, bias_a_1,
           w_self_p_2, w_rel_p_a_2, bias_p_2, w_self_a_2, w_rel_a_p_2, bias_a_2,
           w_self_p_3, w_rel_p_a_3, bias_p_3, w_self_a_3, w_rel_a_p_3, bias_a_3):
    src_pa = adj_p_a[1].reshape(NICH, ICH)
    dst_pa = adj_p_a[0].reshape(NICH, ICH)
    src_ap = adj_a_p[1].reshape(NICH, ICH)
    dst_ap = adj_a_p[0].reshape(NICH, ICH)
    zeros64 = jnp.zeros((ZR * (64 // NS),), jnp.float32)
    zeros16 = jnp.zeros((ZR * (16 // NS),), jnp.float32)

    layers = (
        (w_self_p_0, w_rel_p_a_0, bias_p_0, w_self_a_0, w_rel_a_p_0, bias_a_0),
        (w_self_p_1, w_rel_p_a_1, bias_p_1, w_self_a_1, w_rel_a_p_1, bias_a_1),
        (w_self_p_2, w_rel_p_a_2, bias_p_2, w_self_a_2, w_rel_a_p_2, bias_a_2),
        (w_self_p_3, w_rel_p_a_3, bias_p_3, w_self_a_3, w_rel_a_p_3, bias_a_3),
    )
    sp = sa = nbp = nba = pbias_p = pbias_a = None
    for l, (wsp, wrpa, bp, wsa, wrap, ba) in enumerate(layers):
        if l == 0:
            sp, xwap, sa, xwpa = _tc_mm4(ft_p, ft_a, wsp, wrap, wsa, wrpa)
        else:
            sp, xwap, sa, xwpa = _tc_comb_mm4(
                sp, nbp, pbias_p, sa, nba, pbias_a, wsp, wrap, wsa, wrpa)
        zeros = zeros64 if wsp.shape[1] == 64 else zeros16
        nbp, nba = _sc_spmm(xwpa, xwap, src_pa, dst_pa, src_ap, dst_ap, zeros)
        pbias_p, pbias_a = bp, ba
    return _tc_final(sp, nbp, pbias_p, sa, nba, pbias_a)


# speedup vs baseline: 1.7485x; 1.7485x over previous
import functools

import jax
import jax.numpy as jnp
from jax import lax
from jax.experimental import pallas as pl
from jax.experimental.pallas import tpu as pltpu
from jax.experimental.pallas import tpu_sc as plsc

N_NODE = 10000
E = 320000
NC = 2
NS = 16
ICH = 4000
NICH = E // ICH
GU = 8
ZR = 2000
BM = 2000


def _tc_mm4(xp, xa, wsp, wrap, wsa, wrpa):
    Mrows, K = xp.shape
    N = wsp.shape[1]

    def body(xp_ref, xa_ref, wsp_ref, wrap_ref, wsa_ref, wrpa_ref,
             osp, oxwap, osa, oxwpa):
        xp_b = xp_ref[...]
        xa_b = xa_ref[...]
        osp[...] = jnp.dot(xp_b, wsp_ref[...], preferred_element_type=jnp.float32)
        oxwap[...] = jnp.dot(xp_b, wrap_ref[...], preferred_element_type=jnp.float32)
        osa[...] = jnp.dot(xa_b, wsa_ref[...], preferred_element_type=jnp.float32)
        oxwpa[...] = jnp.dot(xa_b, wrpa_ref[...], preferred_element_type=jnp.float32)

    bs_x = pl.BlockSpec((BM, K), lambda i: (i, 0))
    bs_w = pl.BlockSpec((K, N), lambda i: (0, 0))
    bs_o = pl.BlockSpec((BM, N), lambda i: (i, 0))
    return pl.pallas_call(
        body, grid=(Mrows // BM,),
        in_specs=[bs_x, bs_x, bs_w, bs_w, bs_w, bs_w],
        out_specs=[bs_o, bs_o, bs_o, bs_o],
        out_shape=[jax.ShapeDtypeStruct((Mrows, N), jnp.float32)] * 4,
    )(xp, xa, wsp, wrap, wsa, wrpa)


def _tc_comb_mm4(sp, nbp, bp, sa, nba, ba, wsp, wrap, wsa, wrpa):
    Mrows, K = sp.shape
    N = wsp.shape[1]

    def body(sp_ref, nbp_ref, bp_ref, sa_ref, nba_ref, ba_ref,
             wsp_ref, wrap_ref, wsa_ref, wrpa_ref,
             osp, oxwap, osa, oxwpa):
        xp = (sp_ref[...] + nbp_ref[...]) * 0.5 + bp_ref[...]
        xp = jnp.where(xp > 0, xp, jnp.exp(jnp.minimum(xp, 0.0)) - 1.0)
        xa = (sa_ref[...] + nba_ref[...]) * 0.5 + ba_ref[...]
        xa = jnp.where(xa > 0, xa, jnp.exp(jnp.minimum(xa, 0.0)) - 1.0)
        osp[...] = jnp.dot(xp, wsp_ref[...], preferred_element_type=jnp.float32)
        oxwap[...] = jnp.dot(xp, wrap_ref[...], preferred_element_type=jnp.float32)
        osa[...] = jnp.dot(xa, wsa_ref[...], preferred_element_type=jnp.float32)
        oxwpa[...] = jnp.dot(xa, wrpa_ref[...], preferred_element_type=jnp.float32)

    bs_x = pl.BlockSpec((BM, K), lambda i: (i, 0))
    bs_b = pl.BlockSpec((1, K), lambda i: (0, 0))
    bs_w = pl.BlockSpec((K, N), lambda i: (0, 0))
    bs_o = pl.BlockSpec((BM, N), lambda i: (i, 0))
    return pl.pallas_call(
        body, grid=(Mrows // BM,),
        in_specs=[bs_x, bs_x, bs_b, bs_x, bs_x, bs_b,
                  bs_w, bs_w, bs_w, bs_w],
        out_specs=[bs_o, bs_o, bs_o, bs_o],
        out_shape=[jax.ShapeDtypeStruct((Mrows, N), jnp.float32)] * 4,
    )(sp, nbp, bp, sa, nba, ba, wsp, wrap, wsa, wrpa)


def _tc_final(sp, nbp, bp, sa, nba, ba):
    Mrows, K = sp.shape

    def body(sp_ref, nbp_ref, bp_ref, sa_ref, nba_ref, ba_ref, op, oa):
        op[...] = (sp_ref[...] + nbp_ref[...]) * 0.5 + bp_ref[...]
        oa[...] = (sa_ref[...] + nba_ref[...]) * 0.5 + ba_ref[...]

    bs_x = pl.BlockSpec((BM, K), lambda i: (i, 0))
    bs_b = pl.BlockSpec((1, K), lambda i: (0, 0))
    return pl.pallas_call(
        body, grid=(Mrows // BM,),
        in_specs=[bs_x, bs_x, bs_b, bs_x, bs_x, bs_b],
        out_specs=[bs_x, bs_x],
        out_shape=[jax.ShapeDtypeStruct((Mrows, K), jnp.float32)] * 2,
    )(sp, nbp, bp, sa, nba, ba)


def _sc_spmm(tbl_p, tbl_a, src_pa, dst_pa, src_ap, dst_ap, zeros):
    d = tbl_p.shape[1]
    CG = d // NS
    tbl_p = jnp.transpose(tbl_p.reshape(N_NODE, NS, CG), (1, 0, 2))
    tbl_p = tbl_p.reshape(NS, N_NODE * CG)
    tbl_a = jnp.transpose(tbl_a.reshape(N_NODE, NS, CG), (1, 0, 2))
    tbl_a = tbl_a.reshape(NS, N_NODE * CG)
    mesh = plsc.VectorSubcoreMesh(core_axis_name="c", subcore_axis_name="s")
    out_t = (jax.ShapeDtypeStruct((NS, N_NODE * CG), jnp.float32),
             jax.ShapeDtypeStruct((NS, N_NODE * CG), jnp.float32))

    @functools.partial(
        pl.kernel, mesh=mesh, out_type=out_t,
        compiler_params=pltpu.CompilerParams(
            use_tc_tiling_on_sc=False, needs_layout_passes=False),
        scratch_types=[
            pltpu.VMEM((N_NODE * CG,), jnp.float32),
            pltpu.VMEM((N_NODE * CG,), jnp.float32),
            pltpu.VMEM((2, ICH), jnp.int32),
            pltpu.VMEM((2, ICH), jnp.int32),
            pltpu.SemaphoreType.DMA((2,)),
            pltpu.SemaphoreType.DMA((2,)),
        ],
    )
    def k(tblp_h, tbla_h, spa_h, dpa_h, sap_h, dap_h, zeros_h,
          outp_h, outa_h, tblbuf, accbuf, ibs, ibd, ssem, dsem):
        c = lax.axis_index("c")
        s = lax.axis_index("s")
        zch = ZR * CG
        for kz in range(N_NODE // ZR):
            pltpu.sync_copy(zeros_h, accbuf.at[pl.ds(kz * zch, zch)])

        def run(src_h, dst_h, tbl_h, out_h):
            pltpu.sync_copy(tbl_h.at[s], tblbuf)
            for b in range(2):
                pltpu.async_copy(src_h.at[b], ibs.at[b], ssem.at[b])
                pltpu.async_copy(dst_h.at[b], ibd.at[b], dsem.at[b])
            def outer(t2, _):
                for b in range(2):
                    t = t2 * 2 + b
                    pltpu.make_async_copy(
                        src_h.at[t], ibs.at[b], ssem.at[b]).wait()
                    pltpu.make_async_copy(
                        dst_h.at[t], ibd.at[b], dsem.at[b]).wait()

                    def grp(g, b=b):
                        base = g * 16
                        srcf = ibs[b, pl.ds(base, 16)] * CG
                        dstf = ibd[b, pl.ds(base, 16)] * CG
                        for j in range(CG):
                            v = plsc.load_gather(tblbuf, [srcf + j])
                            plsc.addupdate_scatter(accbuf, [dstf + j], v)

                    plsc.parallel_loop(0, ICH // 16, 1, unroll=GU)(grp)

                    @pl.when(t + 2 < NICH)
                    def _(b=b, t=t):
                        pltpu.async_copy(src_h.at[t + 2], ibs.at[b], ssem.at[b])
                        pltpu.async_copy(dst_h.at[t + 2], ibd.at[b], dsem.at[b])
                return 0

            lax.fori_loop(0, NICH // 2, outer, 0)
            pltpu.sync_copy(accbuf, out_h.at[s])

        @pl.when(c == 0)
        def _():
            run(spa_h, dpa_h, tblp_h, outp_h)

        @pl.when(c == 1)
        def _():
            run(sap_h, dap_h, tbla_h, outa_h)

    nbp, nba = k(tbl_p, tbl_a, src_pa, dst_pa, src_ap, dst_ap, zeros)
    nbp = jnp.transpose(nbp.reshape(NS, N_NODE, CG), (1, 0, 2)).reshape(N_NODE, d)
    nba = jnp.transpose(nba.reshape(NS, N_NODE, CG), (1, 0, 2)).reshape(N_NODE, d)
    return nbp, nba


def kernel(ft_p, ft_a, adj_p_a, adj_a_p,
           w_self_p_0, w_rel_p_a_0, bias_p_0, w_self_a_0, w_rel_a_p_0, bias_a_0,
           w_self_p_1, w_rel_p_a_1, bias_p_1, w_self_a_1, w_rel_a_p_1, bias_a_1,
           w_self_p_2, w_rel_p_a_2, bias_p_2, w_self_a_2, w_rel_a_p_2, bias_a_2,
           w_self_p_3, w_rel_p_a_3, bias_p_3, w_self_a_3, w_rel_a_p_3, bias_a_3):
    src_pa = adj_p_a[1].reshape(NICH, ICH)
    dst_pa = adj_p_a[0].reshape(NICH, ICH)
    src_ap = adj_a_p[1].reshape(NICH, ICH)
    dst_ap = adj_a_p[0].reshape(NICH, ICH)
    zeros64 = jnp.zeros((ZR * (64 // NS),), jnp.float32)
    zeros16 = jnp.zeros((ZR * (16 // NS),), jnp.float32)

    layers = (
        (w_self_p_0, w_rel_p_a_0, bias_p_0, w_self_a_0, w_rel_a_p_0, bias_a_0),
        (w_self_p_1, w_rel_p_a_1, bias_p_1, w_self_a_1, w_rel_a_p_1, bias_a_1),
        (w_self_p_2, w_rel_p_a_2, bias_p_2, w_self_a_2, w_rel_a_p_2, bias_a_2),
        (w_self_p_3, w_rel_p_a_3, bias_p_3, w_self_a_3, w_rel_a_p_3, bias_a_3),
    )
    sp = sa = nbp = nba = pbias_p = pbias_a = None
    for l, (wsp, wrpa, bp, wsa, wrap, ba) in enumerate(layers):
        if l == 0:
            sp, xwap, sa, xwpa = _tc_mm4(ft_p, ft_a, wsp, wrap, wsa, wrpa)
        else:
            sp, xwap, sa, xwpa = _tc_comb_mm4(
                sp, nbp, pbias_p, sa, nba, pbias_a, wsp, wrap, wsa, wrpa)
        zeros = zeros64 if wsp.shape[1] == 64 else zeros16
        nbp, nba = _sc_spmm(xwpa, xwap, src_pa, dst_pa, src_ap, dst_ap, zeros)
        pbias_p, pbias_a = bp, ba
    return _tc_final(sp, nbp, pbias_p, sa, nba, pbias_a)


# speedup vs baseline: 2.8477x; 1.6287x over previous
import functools

import jax
import jax.numpy as jnp
from jax import lax
from jax.experimental import pallas as pl
from jax.experimental.pallas import tpu as pltpu
from jax.experimental.pallas import tpu_sc as plsc

N_NODE = 10000
E = 320000
NC = 2
NS = 16
NW = NC * NS
EPT = E // NW
CH = 40
NCH = EPT // CH
M = 5
D = 2
RSTRIPE = N_NODE // NS
WB = 125
BM = 2000


def _tc_mm4(xp, xa, wsp, wrap, wsa, wrpa):
    Mrows, K = xp.shape
    N = wsp.shape[1]

    def body(xp_ref, xa_ref, wsp_ref, wrap_ref, wsa_ref, wrpa_ref,
             osp, oxwap, osa, oxwpa):
        xp_b = xp_ref[...]
        xa_b = xa_ref[...]
        osp[...] = jnp.dot(xp_b, wsp_ref[...], preferred_element_type=jnp.float32)
        oxwap[...] = jnp.dot(xp_b, wrap_ref[...], preferred_element_type=jnp.float32)
        osa[...] = jnp.dot(xa_b, wsa_ref[...], preferred_element_type=jnp.float32)
        oxwpa[...] = jnp.dot(xa_b, wrpa_ref[...], preferred_element_type=jnp.float32)

    bs_x = pl.BlockSpec((BM, K), lambda i: (i, 0))
    bs_w = pl.BlockSpec((K, N), lambda i: (0, 0))
    bs_o = pl.BlockSpec((BM, N), lambda i: (i, 0))
    return pl.pallas_call(
        body, grid=(Mrows // BM,),
        in_specs=[bs_x, bs_x, bs_w, bs_w, bs_w, bs_w],
        out_specs=[bs_o, bs_o, bs_o, bs_o],
        out_shape=[jax.ShapeDtypeStruct((Mrows, N), jnp.float32)] * 4,
    )(xp, xa, wsp, wrap, wsa, wrpa)


def _tc_comb_mm4(sp, nbp, bp, sa, nba, ba, wsp, wrap, wsa, wrpa):
    Mrows, K = sp.shape
    N = wsp.shape[1]

    def body(sp_ref, nbp_ref, bp_ref, sa_ref, nba_ref, ba_ref,
             wsp_ref, wrap_ref, wsa_ref, wrpa_ref,
             osp, oxwap, osa, oxwpa):
        xp = (sp_ref[...] + nbp_ref[0] + nbp_ref[1]) * 0.5 + bp_ref[...]
        xp = jnp.where(xp > 0, xp, jnp.exp(jnp.minimum(xp, 0.0)) - 1.0)
        xa = (sa_ref[...] + nba_ref[0] + nba_ref[1]) * 0.5 + ba_ref[...]
        xa = jnp.where(xa > 0, xa, jnp.exp(jnp.minimum(xa, 0.0)) - 1.0)
        osp[...] = jnp.dot(xp, wsp_ref[...], preferred_element_type=jnp.float32)
        oxwap[...] = jnp.dot(xp, wrap_ref[...], preferred_element_type=jnp.float32)
        osa[...] = jnp.dot(xa, wsa_ref[...], preferred_element_type=jnp.float32)
        oxwpa[...] = jnp.dot(xa, wrpa_ref[...], preferred_element_type=jnp.float32)

    bs_x = pl.BlockSpec((BM, K), lambda i: (i, 0))
    bs_nb = pl.BlockSpec((NC, BM, K), lambda i: (0, i, 0))
    bs_b = pl.BlockSpec((1, K), lambda i: (0, 0))
    bs_w = pl.BlockSpec((K, N), lambda i: (0, 0))
    bs_o = pl.BlockSpec((BM, N), lambda i: (i, 0))
    return pl.pallas_call(
        body, grid=(Mrows // BM,),
        in_specs=[bs_x, bs_nb, bs_b, bs_x, bs_nb, bs_b,
                  bs_w, bs_w, bs_w, bs_w],
        out_specs=[bs_o, bs_o, bs_o, bs_o],
        out_shape=[jax.ShapeDtypeStruct((Mrows, N), jnp.float32)] * 4,
    )(sp, nbp, bp, sa, nba, ba, wsp, wrap, wsa, wrpa)


def _tc_final(sp, nbp, bp, sa, nba, ba):
    Mrows, K = sp.shape

    def body(sp_ref, nbp_ref, bp_ref, sa_ref, nba_ref, ba_ref, op, oa):
        op[...] = (sp_ref[...] + nbp_ref[0] + nbp_ref[1]) * 0.5 + bp_ref[...]
        oa[...] = (sa_ref[...] + nba_ref[0] + nba_ref[1]) * 0.5 + ba_ref[...]

    bs_x = pl.BlockSpec((BM, K), lambda i: (i, 0))
    bs_nb = pl.BlockSpec((NC, BM, K), lambda i: (0, i, 0))
    bs_b = pl.BlockSpec((1, K), lambda i: (0, 0))
    return pl.pallas_call(
        body, grid=(Mrows // BM,),
        in_specs=[bs_x, bs_nb, bs_b, bs_x, bs_nb, bs_b],
        out_specs=[bs_x, bs_x],
        out_shape=[jax.ShapeDtypeStruct((Mrows, K), jnp.float32)] * 2,
    )(sp, nbp, bp, sa, nba, ba)


def _sc_spmm(tbl_p, tbl_a, src_pa, dst_pa, src_ap, dst_ap, zeros):
    d = tbl_p.shape[1]
    mesh = plsc.VectorSubcoreMesh(core_axis_name="c", subcore_axis_name="s")
    out_t = (jax.ShapeDtypeStruct((NC, NS, RSTRIPE, d), jnp.float32),
             jax.ShapeDtypeStruct((NC, NS, RSTRIPE, d), jnp.float32))

    @functools.partial(
        pl.kernel, mesh=mesh, out_type=out_t,
        compiler_params=pltpu.CompilerParams(use_tc_tiling_on_sc=False),
        scratch_types=[
            pltpu.VMEM_SHARED((N_NODE, d), jnp.float32),
            pltpu.VMEM_SHARED((N_NODE, d), jnp.float32),
            pltpu.VMEM((WB, d), jnp.float32),
            pltpu.VMEM((NCH, CH), jnp.int32),
            pltpu.VMEM((NCH, CH), jnp.int32),
            pltpu.VMEM((M, CH, d), jnp.float32),
            pltpu.SemaphoreType.DMA((M,)),
            pltpu.SemaphoreType.DMA((M,)),
        ],
    )
    def k(tblp_h, tbla_h, srcpa_h, dstpa_h, srcap_h, dstap_h, zeros_h,
          outp_h, outa_h, accp, acca, vbuf, srcb, dstb, rows, gsem, ssem):
        c = lax.axis_index("c")
        s = lax.axis_index("s")
        wid = c * NS + s
        pltpu.sync_copy(zeros_h, vbuf)
        for j in range(RSTRIPE // WB):
            pltpu.sync_copy(vbuf, accp.at[pl.ds(s * RSTRIPE + j * WB, WB)])
            pltpu.sync_copy(vbuf, acca.at[pl.ds(s * RSTRIPE + j * WB, WB)])
        plsc.subcore_barrier()
        for src_h, dst_h, tbl_h, acc in (
            (srcpa_h, dstpa_h, tblp_h, accp),
            (srcap_h, dstap_h, tbla_h, acca),
        ):
            pltpu.sync_copy(src_h.at[wid], srcb)
            pltpu.sync_copy(dst_h.at[wid], dstb)
            for i in range(D):
                pltpu.async_copy(tbl_h.at[srcb.at[i]], rows.at[i], gsem.at[i])

            def outer(go, _, tbl_h=tbl_h, acc=acc):
                for i in range(M):
                    g = go * M + i
                    pltpu.make_async_copy(
                        tbl_h.at[srcb.at[g]], rows.at[i], gsem.at[i]).wait()
                    pltpu.async_copy(
                        rows.at[i], acc.at[dstb.at[g]], ssem.at[i], add=True)
                    sp_ = (i + D) % M
                    pre = g + D

                    @pl.when(jnp.logical_and(pre < NCH, g >= M - D))
                    def _(sp_=sp_, g=g, acc=acc):
                        pltpu.make_async_copy(
                            rows.at[sp_], acc.at[dstb.at[g + D - M]],
                            ssem.at[sp_]).wait()

                    @pl.when(pre < NCH)
                    def _(sp_=sp_, pre=pre, tbl_h=tbl_h):
                        pltpu.async_copy(
                            tbl_h.at[srcb.at[pre]], rows.at[sp_], gsem.at[sp_])
                return 0

            lax.fori_loop(0, NCH // M, outer, 0)
            for j in range(M):
                q = NCH - M + j
                pltpu.make_async_copy(
                    rows.at[q % M], acc.at[dstb.at[q]], ssem.at[q % M]).wait()
        plsc.subcore_barrier()
        for j in range(RSTRIPE // WB):
            row = pl.ds(s * RSTRIPE + j * WB, WB)
            pltpu.sync_copy(accp.at[row], vbuf)
            pltpu.sync_copy(vbuf, outp_h.at[c, s, pl.ds(j * WB, WB)])
            pltpu.sync_copy(acca.at[row], vbuf)
            pltpu.sync_copy(vbuf, outa_h.at[c, s, pl.ds(j * WB, WB)])

    nbp, nba = k(tbl_p, tbl_a, src_pa, dst_pa, src_ap, dst_ap, zeros)
    return (nbp.reshape(NC, N_NODE, d), nba.reshape(NC, N_NODE, d))


def kernel(ft_p, ft_a, adj_p_a, adj_a_p,
           w_self_p_0, w_rel_p_a_0, bias_p_0, w_self_a_0, w_rel_a_p_0, bias_a_0,
           w_self_p_1, w_rel_p_a_1, bias_p_1, w_self_a_1, w_rel_a_p_1, bias_a_1,
           w_self_p_2, w_rel_p_a_2, bias_p_2, w_self_a_2, w_rel_a_p_2, bias_a_2,
           w_self_p_3, w_rel_p_a_3, bias_p_3, w_self_a_3, w_rel_a_p_3, bias_a_3):
    src_pa = adj_p_a[1].reshape(NW, NCH, CH)
    dst_pa = adj_p_a[0].reshape(NW, NCH, CH)
    src_ap = adj_a_p[1].reshape(NW, NCH, CH)
    dst_ap = adj_a_p[0].reshape(NW, NCH, CH)
    zeros64 = jnp.zeros((WB, 64), jnp.float32)
    zeros16 = jnp.zeros((WB, 16), jnp.float32)

    layers = (
        (w_self_p_0, w_rel_p_a_0, bias_p_0, w_self_a_0, w_rel_a_p_0, bias_a_0),
        (w_self_p_1, w_rel_p_a_1, bias_p_1, w_self_a_1, w_rel_a_p_1, bias_a_1),
        (w_self_p_2, w_rel_p_a_2, bias_p_2, w_self_a_2, w_rel_a_p_2, bias_a_2),
        (w_self_p_3, w_rel_p_a_3, bias_p_3, w_self_a_3, w_rel_a_p_3, bias_a_3),
    )
    sp = sa = nbp = nba = pbias_p = pbias_a = None
    for l, (wsp, wrpa, bp, wsa, wrap, ba) in enumerate(layers):
        if l == 0:
            sp, xwap, sa, xwpa = _tc_mm4(ft_p, ft_a, wsp, wrap, wsa, wrpa)
        else:
            sp, xwap, sa, xwpa = _tc_comb_mm4(
                sp, nbp, pbias_p, sa, nba, pbias_a, wsp, wrap, wsa, wrpa)
        zeros = zeros64 if wsp.shape[1] == 64 else zeros16
        nbp, nba = _sc_spmm(xwpa, xwap, src_pa, dst_pa, src_ap, dst_ap, zeros)
        pbias_p, pbias_a = bp, ba
    return _tc_final(sp, nbp, pbias_p, sa, nba, pbias_a)


# speedup vs baseline: 3.6121x; 1.2684x over previous
import functools

import jax
import jax.numpy as jnp
from jax import lax
from jax.experimental import pallas as pl
from jax.experimental.pallas import tpu as pltpu
from jax.experimental.pallas import tpu_sc as plsc

N_NODE = 10000
E = 320000
NC = 2
NS = 16
NW = NC * NS
EPT = E // NW
CH = 40
NCH = EPT // CH
M = 5
D = 3
RSTRIPE = N_NODE // NS
WB = 125
BM = 2000


def _tc_mm4(xp, xa, wsp, wrap, wsa, wrpa):
    Mrows, K = xp.shape
    N = wsp.shape[1]

    def body(xp_ref, xa_ref, wsp_ref, wrap_ref, wsa_ref, wrpa_ref,
             osp, oxwap, osa, oxwpa):
        xp_b = xp_ref[...]
        xa_b = xa_ref[...]
        osp[...] = jnp.dot(xp_b, wsp_ref[...], preferred_element_type=jnp.float32)
        oxwap[...] = jnp.dot(xp_b, wrap_ref[...], preferred_element_type=jnp.float32)
        osa[...] = jnp.dot(xa_b, wsa_ref[...], preferred_element_type=jnp.float32)
        oxwpa[...] = jnp.dot(xa_b, wrpa_ref[...], preferred_element_type=jnp.float32)

    bs_x = pl.BlockSpec((BM, K), lambda i: (i, 0))
    bs_w = pl.BlockSpec((K, N), lambda i: (0, 0))
    bs_o = pl.BlockSpec((BM, N), lambda i: (i, 0))
    return pl.pallas_call(
        body, grid=(Mrows // BM,),
        in_specs=[bs_x, bs_x, bs_w, bs_w, bs_w, bs_w],
        out_specs=[bs_o, bs_o, bs_o, bs_o],
        out_shape=[jax.ShapeDtypeStruct((Mrows, N), jnp.float32)] * 4,
    )(xp, xa, wsp, wrap, wsa, wrpa)


def _tc_comb_mm4(sp, nbp, bp, sa, nba, ba, wsp, wrap, wsa, wrpa):
    Mrows, K = sp.shape
    N = wsp.shape[1]

    def body(sp_ref, nbp_ref, bp_ref, sa_ref, nba_ref, ba_ref,
             wsp_ref, wrap_ref, wsa_ref, wrpa_ref,
             osp, oxwap, osa, oxwpa):
        xp = (sp_ref[...] + nbp_ref[0] + nbp_ref[1]) * 0.5 + bp_ref[...]
        xp = jnp.where(xp > 0, xp, jnp.exp(jnp.minimum(xp, 0.0)) - 1.0)
        xa = (sa_ref[...] + nba_ref[0] + nba_ref[1]) * 0.5 + ba_ref[...]
        xa = jnp.where(xa > 0, xa, jnp.exp(jnp.minimum(xa, 0.0)) - 1.0)
        osp[...] = jnp.dot(xp, wsp_ref[...], preferred_element_type=jnp.float32)
        oxwap[...] = jnp.dot(xp, wrap_ref[...], preferred_element_type=jnp.float32)
        osa[...] = jnp.dot(xa, wsa_ref[...], preferred_element_type=jnp.float32)
        oxwpa[...] = jnp.dot(xa, wrpa_ref[...], preferred_element_type=jnp.float32)

    bs_x = pl.BlockSpec((BM, K), lambda i: (i, 0))
    bs_nb = pl.BlockSpec((NC, BM, K), lambda i: (0, i, 0))
    bs_b = pl.BlockSpec((1, K), lambda i: (0, 0))
    bs_w = pl.BlockSpec((K, N), lambda i: (0, 0))
    bs_o = pl.BlockSpec((BM, N), lambda i: (i, 0))
    return pl.pallas_call(
        body, grid=(Mrows // BM,),
        in_specs=[bs_x, bs_nb, bs_b, bs_x, bs_nb, bs_b,
                  bs_w, bs_w, bs_w, bs_w],
        out_specs=[bs_o, bs_o, bs_o, bs_o],
        out_shape=[jax.ShapeDtypeStruct((Mrows, N), jnp.float32)] * 4,
    )(sp, nbp, bp, sa, nba, ba, wsp, wrap, wsa, wrpa)


def _tc_final(sp, nbp, bp, sa, nba, ba):
    Mrows, K = sp.shape

    def body(sp_ref, nbp_ref, bp_ref, sa_ref, nba_ref, ba_ref, op, oa):
        op[...] = (sp_ref[...] + nbp_ref[0] + nbp_ref[1]) * 0.5 + bp_ref[...]
        oa[...] = (sa_ref[...] + nba_ref[0] + nba_ref[1]) * 0.5 + ba_ref[...]

    bs_x = pl.BlockSpec((BM, K), lambda i: (i, 0))
    bs_nb = pl.BlockSpec((NC, BM, K), lambda i: (0, i, 0))
    bs_b = pl.BlockSpec((1, K), lambda i: (0, 0))
    return pl.pallas_call(
        body, grid=(Mrows // BM,),
        in_specs=[bs_x, bs_nb, bs_b, bs_x, bs_nb, bs_b],
        out_specs=[bs_x, bs_x],
        out_shape=[jax.ShapeDtypeStruct((Mrows, K), jnp.float32)] * 2,
    )(sp, nbp, bp, sa, nba, ba)


def _sc_spmm(tbl_p, tbl_a, src_pa, dst_pa, src_ap, dst_ap, zeros):
    d = tbl_p.shape[1]
    mesh = plsc.VectorSubcoreMesh(core_axis_name="c", subcore_axis_name="s")
    out_t = (jax.ShapeDtypeStruct((NC, NS, RSTRIPE, d), jnp.float32),
             jax.ShapeDtypeStruct((NC, NS, RSTRIPE, d), jnp.float32))

    @functools.partial(
        pl.kernel, mesh=mesh, out_type=out_t,
        compiler_params=pltpu.CompilerParams(use_tc_tiling_on_sc=False),
        scratch_types=[
            pltpu.VMEM_SHARED((N_NODE, d), jnp.float32),
            pltpu.VMEM_SHARED((N_NODE, d), jnp.float32),
            pltpu.VMEM((WB, d), jnp.float32),
            pltpu.VMEM((NCH, CH), jnp.int32),
            pltpu.VMEM((NCH, CH), jnp.int32),
            pltpu.VMEM((M, CH, d), jnp.float32),
            pltpu.SemaphoreType.DMA((M,)),
            pltpu.SemaphoreType.DMA((M,)),
        ],
    )
    def k(tblp_h, tbla_h, srcpa_h, dstpa_h, srcap_h, dstap_h, zeros_h,
          outp_h, outa_h, accp, acca, vbuf, srcb, dstb, rows, gsem, ssem):
        c = lax.axis_index("c")
        s = lax.axis_index("s")
        wid = c * NS + s
        pltpu.sync_copy(zeros_h, vbuf)
        for j in range(RSTRIPE // WB):
            pltpu.sync_copy(vbuf, accp.at[pl.ds(s * RSTRIPE + j * WB, WB)])
            pltpu.sync_copy(vbuf, acca.at[pl.ds(s * RSTRIPE + j * WB, WB)])
        plsc.subcore_barrier()
        for src_h, dst_h, tbl_h, acc in (
            (srcpa_h, dstpa_h, tblp_h, accp),
            (srcap_h, dstap_h, tbla_h, acca),
        ):
            pltpu.sync_copy(src_h.at[wid], srcb)
            pltpu.sync_copy(dst_h.at[wid], dstb)
            for i in range(D):
                pltpu.async_copy(tbl_h.at[srcb.at[i]], rows.at[i], gsem.at[i])

            def outer(go, _, tbl_h=tbl_h, acc=acc):
                for i in range(M):
                    g = go * M + i
                    pltpu.make_async_copy(
                        tbl_h.at[srcb.at[g]], rows.at[i], gsem.at[i]).wait()
                    pltpu.async_copy(
                        rows.at[i], acc.at[dstb.at[g]], ssem.at[i], add=True)
                    sp_ = (i + D) % M
                    pre = g + D

                    @pl.when(jnp.logical_and(pre < NCH, g >= M - D))
                    def _(sp_=sp_, g=g, acc=acc):
                        pltpu.make_async_copy(
                            rows.at[sp_], acc.at[dstb.at[g + D - M]],
                            ssem.at[sp_]).wait()

                    @pl.when(pre < NCH)
                    def _(sp_=sp_, pre=pre, tbl_h=tbl_h):
                        pltpu.async_copy(
                            tbl_h.at[srcb.at[pre]], rows.at[sp_], gsem.at[sp_])
                return 0

            lax.fori_loop(0, NCH // M, outer, 0)
            for j in range(M):
                q = NCH - M + j
                pltpu.make_async_copy(
                    rows.at[q % M], acc.at[dstb.at[q]], ssem.at[q % M]).wait()
        plsc.subcore_barrier()
        for j in range(RSTRIPE // WB):
            row = pl.ds(s * RSTRIPE + j * WB, WB)
            pltpu.sync_copy(accp.at[row], vbuf)
            pltpu.sync_copy(vbuf, outp_h.at[c, s, pl.ds(j * WB, WB)])
            pltpu.sync_copy(acca.at[row], vbuf)
            pltpu.sync_copy(vbuf, outa_h.at[c, s, pl.ds(j * WB, WB)])

    nbp, nba = k(tbl_p, tbl_a, src_pa, dst_pa, src_ap, dst_ap, zeros)
    return (nbp.reshape(NC, N_NODE, d), nba.reshape(NC, N_NODE, d))


def kernel(ft_p, ft_a, adj_p_a, adj_a_p,
           w_self_p_0, w_rel_p_a_0, bias_p_0, w_self_a_0, w_rel_a_p_0, bias_a_0,
           w_self_p_1, w_rel_p_a_1, bias_p_1, w_self_a_1, w_rel_a_p_1, bias_a_1,
           w_self_p_2, w_rel_p_a_2, bias_p_2, w_self_a_2, w_rel_a_p_2, bias_a_2,
           w_self_p_3, w_rel_p_a_3, bias_p_3, w_self_a_3, w_rel_a_p_3, bias_a_3):
    src_pa = adj_p_a[1].reshape(NW, NCH, CH)
    dst_pa = adj_p_a[0].reshape(NW, NCH, CH)
    src_ap = adj_a_p[1].reshape(NW, NCH, CH)
    dst_ap = adj_a_p[0].reshape(NW, NCH, CH)
    zeros64 = jnp.zeros((WB, 64), jnp.float32)
    zeros16 = jnp.zeros((WB, 16), jnp.float32)

    layers = (
        (w_self_p_0, w_rel_p_a_0, bias_p_0, w_self_a_0, w_rel_a_p_0, bias_a_0),
        (w_self_p_1, w_rel_p_a_1, bias_p_1, w_self_a_1, w_rel_a_p_1, bias_a_1),
        (w_self_p_2, w_rel_p_a_2, bias_p_2, w_self_a_2, w_rel_a_p_2, bias_a_2),
        (w_self_p_3, w_rel_p_a_3, bias_p_3, w_self_a_3, w_rel_a_p_3, bias_a_3),
    )
    sp = sa = nbp = nba = pbias_p = pbias_a = None
    for l, (wsp, wrpa, bp, wsa, wrap, ba) in enumerate(layers):
        if l == 0:
            sp, xwap, sa, xwpa = _tc_mm4(ft_p, ft_a, wsp, wrap, wsa, wrpa)
        else:
            sp, xwap, sa, xwpa = _tc_comb_mm4(
                sp, nbp, pbias_p, sa, nba, pbias_a, wsp, wrap, wsa, wrpa)
        zeros = zeros64 if wsp.shape[1] == 64 else zeros16
        nbp, nba = _sc_spmm(xwpa, xwap, src_pa, dst_pa, src_ap, dst_ap, zeros)
        pbias_p, pbias_a = bp, ba
    return _tc_final(sp, nbp, pbias_p, sa, nba, pbias_a)


# speedup vs baseline: 4.0957x; 1.1339x over previous
import functools

import jax
import jax.numpy as jnp
from jax import lax
from jax.experimental import pallas as pl
from jax.experimental.pallas import tpu as pltpu
from jax.experimental.pallas import tpu_sc as plsc

N_NODE = 10000
E = 320000
NC = 2
NS = 16
NW = NC * NS
EPT = E // NW
CH = 40
NCH = EPT // CH
M = 5
D = 4
RSTRIPE = N_NODE // NS
WB = 125
BM = 2000


def _tc_mm4(xp, xa, wsp, wrap, wsa, wrpa):
    Mrows, K = xp.shape
    N = wsp.shape[1]

    def body(xp_ref, xa_ref, wsp_ref, wrap_ref, wsa_ref, wrpa_ref,
             osp, oxwap, osa, oxwpa):
        xp_b = xp_ref[...]
        xa_b = xa_ref[...]
        osp[...] = jnp.dot(xp_b, wsp_ref[...], preferred_element_type=jnp.float32)
        oxwap[...] = jnp.dot(xp_b, wrap_ref[...], preferred_element_type=jnp.float32)
        osa[...] = jnp.dot(xa_b, wsa_ref[...], preferred_element_type=jnp.float32)
        oxwpa[...] = jnp.dot(xa_b, wrpa_ref[...], preferred_element_type=jnp.float32)

    bs_x = pl.BlockSpec((BM, K), lambda i: (i, 0))
    bs_w = pl.BlockSpec((K, N), lambda i: (0, 0))
    bs_o = pl.BlockSpec((BM, N), lambda i: (i, 0))
    return pl.pallas_call(
        body, grid=(Mrows // BM,),
        in_specs=[bs_x, bs_x, bs_w, bs_w, bs_w, bs_w],
        out_specs=[bs_o, bs_o, bs_o, bs_o],
        out_shape=[jax.ShapeDtypeStruct((Mrows, N), jnp.float32)] * 4,
    )(xp, xa, wsp, wrap, wsa, wrpa)


def _tc_comb_mm4(sp, nbp, bp, sa, nba, ba, wsp, wrap, wsa, wrpa):
    Mrows, K = sp.shape
    N = wsp.shape[1]

    def body(sp_ref, nbp_ref, bp_ref, sa_ref, nba_ref, ba_ref,
             wsp_ref, wrap_ref, wsa_ref, wrpa_ref,
             osp, oxwap, osa, oxwpa):
        xp = (sp_ref[...] + nbp_ref[0] + nbp_ref[1]) * 0.5 + bp_ref[...]
        xp = jnp.where(xp > 0, xp, jnp.exp(jnp.minimum(xp, 0.0)) - 1.0)
        xa = (sa_ref[...] + nba_ref[0] + nba_ref[1]) * 0.5 + ba_ref[...]
        xa = jnp.where(xa > 0, xa, jnp.exp(jnp.minimum(xa, 0.0)) - 1.0)
        osp[...] = jnp.dot(xp, wsp_ref[...], preferred_element_type=jnp.float32)
        oxwap[...] = jnp.dot(xp, wrap_ref[...], preferred_element_type=jnp.float32)
        osa[...] = jnp.dot(xa, wsa_ref[...], preferred_element_type=jnp.float32)
        oxwpa[...] = jnp.dot(xa, wrpa_ref[...], preferred_element_type=jnp.float32)

    bs_x = pl.BlockSpec((BM, K), lambda i: (i, 0))
    bs_nb = pl.BlockSpec((NC, BM, K), lambda i: (0, i, 0))
    bs_b = pl.BlockSpec((1, K), lambda i: (0, 0))
    bs_w = pl.BlockSpec((K, N), lambda i: (0, 0))
    bs_o = pl.BlockSpec((BM, N), lambda i: (i, 0))
    return pl.pallas_call(
        body, grid=(Mrows // BM,),
        in_specs=[bs_x, bs_nb, bs_b, bs_x, bs_nb, bs_b,
                  bs_w, bs_w, bs_w, bs_w],
        out_specs=[bs_o, bs_o, bs_o, bs_o],
        out_shape=[jax.ShapeDtypeStruct((Mrows, N), jnp.float32)] * 4,
    )(sp, nbp, bp, sa, nba, ba, wsp, wrap, wsa, wrpa)


def _tc_final(sp, nbp, bp, sa, nba, ba):
    Mrows, K = sp.shape

    def body(sp_ref, nbp_ref, bp_ref, sa_ref, nba_ref, ba_ref, op, oa):
        op[...] = (sp_ref[...] + nbp_ref[0] + nbp_ref[1]) * 0.5 + bp_ref[...]
        oa[...] = (sa_ref[...] + nba_ref[0] + nba_ref[1]) * 0.5 + ba_ref[...]

    bs_x = pl.BlockSpec((BM, K), lambda i: (i, 0))
    bs_nb = pl.BlockSpec((NC, BM, K), lambda i: (0, i, 0))
    bs_b = pl.BlockSpec((1, K), lambda i: (0, 0))
    return pl.pallas_call(
        body, grid=(Mrows // BM,),
        in_specs=[bs_x, bs_nb, bs_b, bs_x, bs_nb, bs_b],
        out_specs=[bs_x, bs_x],
        out_shape=[jax.ShapeDtypeStruct((Mrows, K), jnp.float32)] * 2,
    )(sp, nbp, bp, sa, nba, ba)


def _sc_spmm(tbl_p, tbl_a, src_pa, dst_pa, src_ap, dst_ap, zeros):
    d = tbl_p.shape[1]
    mesh = plsc.VectorSubcoreMesh(core_axis_name="c", subcore_axis_name="s")
    out_t = (jax.ShapeDtypeStruct((NC, NS, RSTRIPE, d), jnp.float32),
             jax.ShapeDtypeStruct((NC, NS, RSTRIPE, d), jnp.float32))

    @functools.partial(
        pl.kernel, mesh=mesh, out_type=out_t,
        compiler_params=pltpu.CompilerParams(use_tc_tiling_on_sc=False),
        scratch_types=[
            pltpu.VMEM_SHARED((N_NODE, d), jnp.float32),
            pltpu.VMEM_SHARED((N_NODE, d), jnp.float32),
            pltpu.VMEM((WB, d), jnp.float32),
            pltpu.VMEM((NCH, CH), jnp.int32),
            pltpu.VMEM((NCH, CH), jnp.int32),
            pltpu.VMEM((M, CH, d), jnp.float32),
            pltpu.SemaphoreType.DMA((M,)),
            pltpu.SemaphoreType.DMA((M,)),
        ],
    )
    def k(tblp_h, tbla_h, srcpa_h, dstpa_h, srcap_h, dstap_h, zeros_h,
          outp_h, outa_h, accp, acca, vbuf, srcb, dstb, rows, gsem, ssem):
        c = lax.axis_index("c")
        s = lax.axis_index("s")
        wid = c * NS + s
        pltpu.sync_copy(zeros_h, vbuf)
        for j in range(RSTRIPE // WB):
            pltpu.sync_copy(vbuf, accp.at[pl.ds(s * RSTRIPE + j * WB, WB)])
            pltpu.sync_copy(vbuf, acca.at[pl.ds(s * RSTRIPE + j * WB, WB)])
        plsc.subcore_barrier()
        for src_h, dst_h, tbl_h, acc in (
            (srcpa_h, dstpa_h, tblp_h, accp),
            (srcap_h, dstap_h, tbla_h, acca),
        ):
            pltpu.sync_copy(src_h.at[wid], srcb)
            pltpu.sync_copy(dst_h.at[wid], dstb)
            for i in range(D):
                pltpu.async_copy(tbl_h.at[srcb.at[i]], rows.at[i], gsem.at[i])

            def outer(go, _, tbl_h=tbl_h, acc=acc):
                for i in range(M):
                    g = go * M + i
                    pltpu.make_async_copy(
                        tbl_h.at[srcb.at[g]], rows.at[i], gsem.at[i]).wait()
                    pltpu.async_copy(
                        rows.at[i], acc.at[dstb.at[g]], ssem.at[i], add=True)
                    sp_ = (i + D) % M
                    pre = g + D

                    @pl.when(jnp.logical_and(pre < NCH, g >= M - D))
                    def _(sp_=sp_, g=g, acc=acc):
                        pltpu.make_async_copy(
                            rows.at[sp_], acc.at[dstb.at[g + D - M]],
                            ssem.at[sp_]).wait()

                    @pl.when(pre < NCH)
                    def _(sp_=sp_, pre=pre, tbl_h=tbl_h):
                        pltpu.async_copy(
                            tbl_h.at[srcb.at[pre]], rows.at[sp_], gsem.at[sp_])
                return 0

            lax.fori_loop(0, NCH // M, outer, 0)
            for j in range(M):
                q = NCH - M + j
                pltpu.make_async_copy(
                    rows.at[q % M], acc.at[dstb.at[q]], ssem.at[q % M]).wait()
        plsc.subcore_barrier()
        for j in range(RSTRIPE // WB):
            row = pl.ds(s * RSTRIPE + j * WB, WB)
            pltpu.sync_copy(accp.at[row], vbuf)
            pltpu.sync_copy(vbuf, outp_h.at[c, s, pl.ds(j * WB, WB)])
            pltpu.sync_copy(acca.at[row], vbuf)
            pltpu.sync_copy(vbuf, outa_h.at[c, s, pl.ds(j * WB, WB)])

    nbp, nba = k(tbl_p, tbl_a, src_pa, dst_pa, src_ap, dst_ap, zeros)
    return (nbp.reshape(NC, N_NODE, d), nba.reshape(NC, N_NODE, d))


def kernel(ft_p, ft_a, adj_p_a, adj_a_p,
           w_self_p_0, w_rel_p_a_0, bias_p_0, w_self_a_0, w_rel_a_p_0, bias_a_0,
           w_self_p_1, w_rel_p_a_1, bias_p_1, w_self_a_1, w_rel_a_p_1, bias_a_1,
           w_self_p_2, w_rel_p_a_2, bias_p_2, w_self_a_2, w_rel_a_p_2, bias_a_2,
           w_self_p_3, w_rel_p_a_3, bias_p_3, w_self_a_3, w_rel_a_p_3, bias_a_3):
    src_pa = adj_p_a[1].reshape(NW, NCH, CH)
    dst_pa = adj_p_a[0].reshape(NW, NCH, CH)
    src_ap = adj_a_p[1].reshape(NW, NCH, CH)
    dst_ap = adj_a_p[0].reshape(NW, NCH, CH)
    zeros64 = jnp.zeros((WB, 64), jnp.float32)
    zeros16 = jnp.zeros((WB, 16), jnp.float32)

    layers = (
        (w_self_p_0, w_rel_p_a_0, bias_p_0, w_self_a_0, w_rel_a_p_0, bias_a_0),
        (w_self_p_1, w_rel_p_a_1, bias_p_1, w_self_a_1, w_rel_a_p_1, bias_a_1),
        (w_self_p_2, w_rel_p_a_2, bias_p_2, w_self_a_2, w_rel_a_p_2, bias_a_2),
        (w_self_p_3, w_rel_p_a_3, bias_p_3, w_self_a_3, w_rel_a_p_3, bias_a_3),
    )
    sp = sa = nbp = nba = pbias_p = pbias_a = None
    for l, (wsp, wrpa, bp, wsa, wrap, ba) in enumerate(layers):
        if l == 0:
            sp, xwap, sa, xwpa = _tc_mm4(ft_p, ft_a, wsp, wrap, wsa, wrpa)
        else:
            sp, xwap, sa, xwpa = _tc_comb_mm4(
                sp, nbp, pbias_p, sa, nba, pbias_a, wsp, wrap, wsa, wrpa)
        zeros = zeros64 if wsp.shape[1] == 64 else zeros16
        nbp, nba = _sc_spmm(xwpa, xwap, src_pa, dst_pa, src_ap, dst_ap, zeros)
        pbias_p, pbias_a = bp, ba
    return _tc_final(sp, nbp, pbias_p, sa, nba, pbias_a)
